# baseline (speedup 1.0000x reference)
import functools

import jax
import jax.numpy as jnp
from jax import lax
from jax.experimental import pallas as pl
from jax.experimental.pallas import tpu as pltpu
from jax.experimental.pallas import tpu_sc as plsc

VOCAB = 361
DIM = 512
NC, NS = 2, 16
NW = NC * NS
NSEG = 4
NPATH = 1024
PLEN = 50
PMAIN = 48
PPAD = 64
WPS = NW // NSEG
PPW = NPATH // WPS
TGRP = 8
NTG = PPW // TGRP

_mesh = plsc.VectorSubcoreMesh(core_axis_name="c", subcore_axis_name="s")


@functools.partial(
    pl.kernel,
    mesh=_mesh,
    out_type=(
        [jax.ShapeDtypeStruct((NPATH, PLEN, DIM), jnp.float32) for _ in range(NSEG)]
        + [jax.ShapeDtypeStruct((NSEG * NPATH, DIM), jnp.float32) for _ in range(2)]
    ),
    scratch_types=[
        pltpu.VMEM((PPW, PPAD), jnp.int32),
        pltpu.VMEM((PPW,), jnp.int32),
        pltpu.VMEM((PPW,), jnp.int32),
        pltpu.VMEM((PLEN, DIM), jnp.float32),
        pltpu.VMEM((PLEN, DIM), jnp.float32),
        pltpu.VMEM((TGRP, DIM), jnp.float32),
        pltpu.VMEM((TGRP, DIM), jnp.float32),
        pltpu.SemaphoreType.DMA,
        pltpu.SemaphoreType.DMA,
        pltpu.SemaphoreType.DMA,
        pltpu.SemaphoreType.DMA,
    ],
)
def _gather(idx_hbm, t48_hbm, t49_hbm, table_hbm, o0, o1, o2, o3, t48, t49,
            idx_v, i48_v, i49_v, buf0, buf1, tb0, tb1, g0, g1, s0, s1):
    wid = lax.axis_index("s") * NC + lax.axis_index("c")
    pltpu.sync_copy(idx_hbm.at[pl.ds(wid * PPW, PPW)], idx_v)
    pltpu.sync_copy(t48_hbm.at[pl.ds(wid * PPW, PPW)], i48_v)
    pltpu.sync_copy(t49_hbm.at[pl.ds(wid * PPW, PPW)], i49_v)

    def gather_start(p, buf, sem):
        pltpu.make_async_copy(
            table_hbm.at[idx_v.at[p].at[pl.ds(0, PLEN)]], buf, sem
        ).start()

    def gather_wait(p, buf, sem):
        pltpu.make_async_copy(
            table_hbm.at[idx_v.at[p].at[pl.ds(0, PLEN)]], buf, sem
        ).wait()

    for seg, out_hbm in enumerate((o0, o1, o2, o3)):

        @pl.when(wid // WPS == seg)
        def _():
            base = (wid - seg * WPS) * PPW

            def scatter_start(p, buf, sem):
                pltpu.make_async_copy(
                    buf.at[pl.ds(0, PMAIN)],
                    out_hbm.at[base + p].at[pl.ds(0, PMAIN)],
                    sem,
                ).start()

            def scatter_wait(p, buf, sem):
                pltpu.make_async_copy(
                    buf.at[pl.ds(0, PMAIN)],
                    out_hbm.at[base + p].at[pl.ds(0, PMAIN)],
                    sem,
                ).wait()

            gather_start(0, buf0, g0)
            gather_start(1, buf1, g1)

            def body(g, carry):
                p0 = 2 * g
                p1 = p0 + 1
                gather_wait(p0, buf0, g0)
                scatter_start(p0, buf0, s0)
                gather_wait(p1, buf1, g1)
                scatter_start(p1, buf1, s1)
                scatter_wait(p0, buf0, s0)
                gather_start(p0 + 2, buf0, g0)
                scatter_wait(p1, buf1, s1)
                gather_start(p1 + 2, buf1, g1)
                return carry

            lax.fori_loop(0, PPW // 2 - 1, body, 0)

            p0 = PPW - 2
            p1 = PPW - 1
            gather_wait(p0, buf0, g0)
            scatter_start(p0, buf0, s0)
            gather_wait(p1, buf1, g1)
            scatter_start(p1, buf1, s1)
            scatter_wait(p0, buf0, s0)
            scatter_wait(p1, buf1, s1)

    tbase = wid * PPW

    def tloop(iv, tout, tb, gsem, ssem):
        def tg_start(k):
            pltpu.make_async_copy(
                table_hbm.at[iv.at[pl.ds(k * TGRP, TGRP)]], tb, gsem
            ).start()

        def tg_wait(k):
            pltpu.make_async_copy(
                table_hbm.at[iv.at[pl.ds(k * TGRP, TGRP)]], tb, gsem
            ).wait()

        def ts_start(k):
            pltpu.make_async_copy(
                tb, tout.at[pl.ds(tbase + k * TGRP, TGRP)], ssem
            ).start()

        def ts_wait(k):
            pltpu.make_async_copy(
                tb, tout.at[pl.ds(tbase + k * TGRP, TGRP)], ssem
            ).wait()

        tg_start(0)

        def tbody(k, carry):
            tg_wait(k)
            ts_start(k)
            ts_wait(k)
            tg_start(k + 1)
            return carry

        lax.fori_loop(0, NTG - 1, tbody, 0)
        tg_wait(NTG - 1)
        ts_start(NTG - 1)
        ts_wait(NTG - 1)

    tloop(i48_v, t48, tb0, g0, s0)
    tloop(i49_v, t49, tb1, g1, s1)


_MRG = 64
_NMB = NPATH // _MRG


def _merge(seg):
    def body(main_ref, t48_ref, t49_ref, out_ref):
        out_ref[:, pl.ds(0, 1), :] = t48_ref[...][:, None, :]
        out_ref[:, pl.ds(1, 1), :] = t49_ref[...][:, None, :]

    return pl.pallas_call(
        body,
        grid=(_NMB,),
        in_specs=[
            pl.BlockSpec(memory_space=pl.ANY),
            pl.BlockSpec((_MRG, DIM), lambda g: (seg * _NMB + g, 0)),
            pl.BlockSpec((_MRG, DIM), lambda g: (seg * _NMB + g, 0)),
        ],
        out_specs=pl.BlockSpec((_MRG, 8, DIM), lambda g: (g, PMAIN // 8, 0)),
        out_shape=jax.ShapeDtypeStruct((NPATH, PLEN, DIM), jnp.float32),
        input_output_aliases={0: 0},
    )


def kernel(path, table):
    idx = path.reshape(NSEG * NPATH, PLEN).astype(jnp.int32)
    idx_pad = jnp.pad(idx, ((0, 0), (0, PPAD - PLEN)))
    o0, o1, o2, o3, t48, t49 = _gather(
        idx_pad, idx[:, PMAIN], idx[:, PMAIN + 1], table
    )
    return tuple(
        _merge(seg)(o, t48, t49) for seg, o in enumerate((o0, o1, o2, o3))
    )


# speedup vs baseline: 1.0798x; 1.0798x over previous
import functools

import jax
import jax.numpy as jnp
from jax import lax
from jax.experimental import pallas as pl
from jax.experimental.pallas import tpu as pltpu
from jax.experimental.pallas import tpu_sc as plsc

VOCAB = 361
DIM = 512
NC, NS = 2, 16
NW = NC * NS
NSEG = 4
NPATH = 1024
PLEN = 50
PMAIN = 48
PTAIL = PLEN - PMAIN
PPAD = 64
WPS = NW // NSEG
PPW = NPATH // WPS
GRP = 8 // PTAIL
NGRP = PPW // GRP

_mesh = plsc.VectorSubcoreMesh(core_axis_name="c", subcore_axis_name="s")


@functools.partial(
    pl.kernel,
    mesh=_mesh,
    out_type=(
        [jax.ShapeDtypeStruct((NPATH, PLEN, DIM), jnp.float32) for _ in range(NSEG)]
        + [jax.ShapeDtypeStruct((NSEG * NPATH * PTAIL, DIM), jnp.float32)]
    ),
    scratch_types=[
        pltpu.VMEM((PPW, PPAD), jnp.int32),
        pltpu.VMEM((NGRP * 8,), jnp.int32),
        pltpu.VMEM((PLEN, DIM), jnp.float32),
        pltpu.VMEM((PLEN, DIM), jnp.float32),
        pltpu.VMEM((8, DIM), jnp.float32),
        pltpu.VMEM((8, DIM), jnp.float32),
        pltpu.SemaphoreType.DMA,
        pltpu.SemaphoreType.DMA,
        pltpu.SemaphoreType.DMA,
        pltpu.SemaphoreType.DMA,
    ],
)
def _gather(idx_hbm, tidx_hbm, table_hbm, o0, o1, o2, o3, tails,
            idx_v, tidx_v, buf0, buf1, tb0, tb1, g0, g1, s0, s1):
    wid = lax.axis_index("s") * NC + lax.axis_index("c")
    pltpu.sync_copy(idx_hbm.at[pl.ds(wid * PPW, PPW)], idx_v)
    pltpu.sync_copy(tidx_hbm.at[pl.ds(wid * NGRP * 8, NGRP * 8)], tidx_v)

    def gather_start(p, buf, sem):
        pltpu.make_async_copy(
            table_hbm.at[idx_v.at[p].at[pl.ds(0, PLEN)]], buf, sem
        ).start()

    def gather_wait(p, buf, sem):
        pltpu.make_async_copy(
            table_hbm.at[idx_v.at[p].at[pl.ds(0, PLEN)]], buf, sem
        ).wait()

    for seg, out_hbm in enumerate((o0, o1, o2, o3)):

        @pl.when(wid // WPS == seg)
        def _():
            base = (wid - seg * WPS) * PPW

            def scatter_start(p, buf, sem):
                pltpu.make_async_copy(
                    buf.at[pl.ds(0, PMAIN)],
                    out_hbm.at[base + p].at[pl.ds(0, PMAIN)],
                    sem,
                ).start()

            def scatter_wait(p, buf, sem):
                pltpu.make_async_copy(
                    buf.at[pl.ds(0, PMAIN)],
                    out_hbm.at[base + p].at[pl.ds(0, PMAIN)],
                    sem,
                ).wait()

            gather_start(0, buf0, g0)
            gather_start(1, buf1, g1)

            def body(g, carry):
                p0 = 2 * g
                p1 = p0 + 1
                gather_wait(p0, buf0, g0)
                scatter_start(p0, buf0, s0)
                gather_wait(p1, buf1, g1)
                scatter_start(p1, buf1, s1)
                scatter_wait(p0, buf0, s0)
                gather_start(p0 + 2, buf0, g0)
                scatter_wait(p1, buf1, s1)
                gather_start(p1 + 2, buf1, g1)
                return carry

            lax.fori_loop(0, PPW // 2 - 1, body, 0)

            p0 = PPW - 2
            p1 = PPW - 1
            gather_wait(p0, buf0, g0)
            scatter_start(p0, buf0, s0)
            gather_wait(p1, buf1, g1)
            scatter_start(p1, buf1, s1)
            scatter_wait(p0, buf0, s0)
            scatter_wait(p1, buf1, s1)

    tbase = wid * NGRP * 8

    def tg_start(k, buf, sem):
        pltpu.make_async_copy(
            table_hbm.at[tidx_v.at[pl.ds(k * 8, 8)]], buf, sem
        ).start()

    def tg_wait(k, buf, sem):
        pltpu.make_async_copy(
            table_hbm.at[tidx_v.at[pl.ds(k * 8, 8)]], buf, sem
        ).wait()

    def ts_start(k, buf, sem):
        pltpu.make_async_copy(buf, tails.at[pl.ds(tbase + k * 8, 8)], sem).start()

    def ts_wait(k, buf, sem):
        pltpu.make_async_copy(buf, tails.at[pl.ds(tbase + k * 8, 8)], sem).wait()

    tg_start(0, tb0, g0)
    tg_start(1, tb1, g1)

    def tbody(g, carry):
        k0 = 2 * g
        k1 = k0 + 1
        tg_wait(k0, tb0, g0)
        ts_start(k0, tb0, s0)
        tg_wait(k1, tb1, g1)
        ts_start(k1, tb1, s1)
        ts_wait(k0, tb0, s0)
        tg_start(k0 + 2, tb0, g0)
        ts_wait(k1, tb1, s1)
        tg_start(k1 + 2, tb1, g1)
        return carry

    lax.fori_loop(0, NGRP // 2 - 1, tbody, 0)
    k0 = NGRP - 2
    k1 = NGRP - 1
    tg_wait(k0, tb0, g0)
    ts_start(k0, tb0, s0)
    tg_wait(k1, tb1, g1)
    ts_start(k1, tb1, s1)
    ts_wait(k0, tb0, s0)
    ts_wait(k1, tb1, s1)


def kernel(path, table):
    idx = path.reshape(NSEG * NPATH, PLEN).astype(jnp.int32)
    idx_pad = jnp.pad(idx, ((0, 0), (0, PPAD - PLEN)))
    tidx = idx[:, PMAIN:].reshape(-1)
    *outs, tails = _gather(idx_pad, tidx, table)
    tails = tails.reshape(NSEG, NPATH, PTAIL, DIM)
    return tuple(
        lax.dynamic_update_slice(o, tails[i], (0, PMAIN, 0))
        for i, o in enumerate(outs)
    )
